# Initial kernel scaffold; baseline (speedup 1.0000x reference)
import jax
import jax.numpy as jnp
from jax.experimental import pallas as pl


def kernel(h, edge_index, edge_attr, node_mask, edge_mask, W_e1, b_e1, W_e2, b_e2, W_att, b_att, W_n1, b_n1, W_n2, b_n2):
    raise NotImplementedError("write your pallas kernel here")


# speedup vs baseline: 2.4938x; 2.4938x over previous
import functools

import jax
import jax.numpy as jnp
from jax import lax
from jax.experimental import pallas as pl
from jax.experimental.pallas import tpu as pltpu
from jax.experimental.pallas import tpu_sc as plsc

NC = 2
NS = 16
NW = NC * NS
LANES = 16
CHUNK = 128
TAIL = 16


def _silu(x):
    return x * jax.nn.sigmoid(x)


def _pre_body(h_ref, wa_ref, wb_ref, ha_ref, hb_ref):
    h = h_ref[...]
    ha_ref[...] = jnp.dot(h, wa_ref[...], preferred_element_type=jnp.float32)
    hb_ref[...] = jnp.dot(h, wb_ref[...], preferred_element_type=jnp.float32)


def _tc_pre(h, WeA, WeB):
    n, d = h.shape
    bn = 2000
    grid = (n // bn,)
    return pl.pallas_call(
        _pre_body,
        grid=grid,
        in_specs=[
            pl.BlockSpec((bn, d), lambda i: (i, 0)),
            pl.BlockSpec(WeA.shape, lambda i: (0, 0)),
            pl.BlockSpec(WeB.shape, lambda i: (0, 0)),
        ],
        out_specs=[
            pl.BlockSpec((bn, WeA.shape[1]), lambda i: (i, 0)),
            pl.BlockSpec((bn, WeB.shape[1]), lambda i: (i, 0)),
        ],
        out_shape=[
            jax.ShapeDtypeStruct((n, WeA.shape[1]), jnp.float32),
            jax.ShapeDtypeStruct((n, WeB.shape[1]), jnp.float32),
        ],
    )(h, WeA, WeB)


def _sc_gather_add(hA, hB, rows, cols):
    n, hdim = hA.shape
    e = rows.shape[0]
    ew = e // NW
    nfull = ew // CHUNK
    rem = ew - nfull * CHUNK
    mesh = plsc.VectorSubcoreMesh(
        core_axis_name="c", subcore_axis_name="s", num_cores=NC, num_subcores=NS)

    @functools.partial(
        pl.kernel,
        out_type=jax.ShapeDtypeStruct((e, hdim), jnp.float32),
        mesh=mesh,
        scratch_types=[
            pltpu.VMEM((CHUNK,), jnp.int32),
            pltpu.VMEM((CHUNK,), jnp.int32),
            pltpu.VMEM((CHUNK, hdim), jnp.float32),
            pltpu.VMEM((CHUNK, hdim), jnp.float32),
            pltpu.VMEM((TAIL,), jnp.int32),
            pltpu.VMEM((TAIL,), jnp.int32),
            pltpu.VMEM((TAIL, hdim), jnp.float32),
            pltpu.VMEM((TAIL, hdim), jnp.float32),
            pltpu.SemaphoreType.DMA,
            pltpu.SemaphoreType.DMA,
        ],
    )
    def gather_kernel(ha_hbm, hb_hbm, rows_hbm, cols_hbm, out_hbm,
                      ir, ic, ba, bb, irt, ict, bat, bbt, sa, sb):
        wid = lax.axis_index("s") * NC + lax.axis_index("c")
        base0 = wid * ew

        def do_chunk(base, idxr, idxc, bufa, bufb, k):
            pltpu.sync_copy(rows_hbm.at[pl.ds(base, k)], idxr)
            pltpu.sync_copy(cols_hbm.at[pl.ds(base, k)], idxc)
            cpa = pltpu.async_copy(ha_hbm.at[idxr], bufa, sa)
            cpb = pltpu.async_copy(hb_hbm.at[idxc], bufb, sb)
            cpa.wait()
            cpb.wait()

            def addrow(i, _):
                for j in range(hdim // LANES):
                    sl = pl.ds(j * LANES, LANES)
                    bufa[i, sl] = bufa[i, sl] + bufb[i, sl]
                return 0

            lax.fori_loop(0, k, addrow, 0)
            pltpu.sync_copy(bufa, out_hbm.at[pl.ds(base, k)])

        def chunk_loop(c, _):
            do_chunk(base0 + c * CHUNK, ir, ic, ba, bb, CHUNK)
            return 0

        lax.fori_loop(0, nfull, chunk_loop, 0)
        if rem:
            do_chunk(base0 + nfull * CHUNK, irt, ict, bat, bbt, rem)

    return gather_kernel(hA, hB, rows, cols)


def _edge_body(pre_ref, ea_ref, em_ref, wc_ref, b1_ref, w2_ref, b2_ref,
               wa_ref, ba_ref, mij_ref, ef_ref):
    x = pre_ref[...] + jnp.dot(ea_ref[...], wc_ref[...],
                               preferred_element_type=jnp.float32) + b1_ref[...]
    m = _silu(x)
    y = jnp.dot(m, w2_ref[...], preferred_element_type=jnp.float32) + b2_ref[...]
    mij = _silu(y)
    att = jax.nn.sigmoid(
        jnp.sum(mij * wa_ref[...], axis=1, keepdims=True) + ba_ref[0, 0])
    mij_ref[...] = mij
    ef_ref[...] = mij * (att * em_ref[...])


def _tc_edge(pre1, edge_attr, edge_mask, WeC, b_e1, W_e2, b_e2, W_att, b_att):
    e, hdim = pre1.shape
    de = edge_attr.shape[1]
    be = 512
    grid = (e // be,)
    wa_row = W_att.reshape(1, hdim)
    ba = b_att.reshape(1, 1)
    b1 = b_e1.reshape(1, hdim)
    b2 = b_e2.reshape(1, hdim)
    return pl.pallas_call(
        _edge_body,
        grid=grid,
        in_specs=[
            pl.BlockSpec((be, hdim), lambda i: (i, 0)),
            pl.BlockSpec((be, de), lambda i: (i, 0)),
            pl.BlockSpec((be, 1), lambda i: (i, 0)),
            pl.BlockSpec((de, hdim), lambda i: (0, 0)),
            pl.BlockSpec((1, hdim), lambda i: (0, 0)),
            pl.BlockSpec((hdim, hdim), lambda i: (0, 0)),
            pl.BlockSpec((1, hdim), lambda i: (0, 0)),
            pl.BlockSpec((1, hdim), lambda i: (0, 0)),
            pl.BlockSpec((1, 1), lambda i: (0, 0)),
        ],
        out_specs=[
            pl.BlockSpec((be, hdim), lambda i: (i, 0)),
            pl.BlockSpec((be, hdim), lambda i: (i, 0)),
        ],
        out_shape=[
            jax.ShapeDtypeStruct((e, hdim), jnp.float32),
            jax.ShapeDtypeStruct((e, hdim), jnp.float32),
        ],
    )(pre1, edge_attr, edge_mask, WeC, b1, W_e2, b2, wa_row, ba)


def _sc_scatter(ef, rows, n):
    e, hdim = ef.shape
    ew = e // NW
    nfull = ew // CHUNK
    rem = ew - nfull * CHUNK
    slab = ((n + NS - 1) // NS + 7) // 8 * 8
    slab_last = n - slab * (NS - 1)
    assert slab_last > 0
    mesh = plsc.VectorSubcoreMesh(
        core_axis_name="c", subcore_axis_name="s", num_cores=NC, num_subcores=NS)
    zeros = jnp.zeros((slab, hdim), jnp.float32)

    @functools.partial(
        pl.kernel,
        out_type=jax.ShapeDtypeStruct((NC, n, hdim), jnp.float32),
        mesh=mesh,
        scratch_types=[
            pltpu.VMEM((CHUNK,), jnp.int32),
            pltpu.VMEM((CHUNK, hdim), jnp.float32),
            pltpu.VMEM((TAIL,), jnp.int32),
            pltpu.VMEM((TAIL, hdim), jnp.float32),
            pltpu.VMEM_SHARED((n, hdim), jnp.float32),
        ],
    )
    def scatter_kernel(ef_hbm, rows_hbm, z_hbm, out_hbm, ir, buf, irt, buft, acc_sh):
        cid = lax.axis_index("c")
        sid = lax.axis_index("s")
        wid = sid * NC + cid
        base0 = wid * ew

        @pl.when(sid < NS - 1)
        def _():
            pltpu.sync_copy(z_hbm, acc_sh.at[pl.ds(sid * slab, slab)])

        @pl.when(sid == NS - 1)
        def _():
            pltpu.sync_copy(z_hbm.at[pl.ds(0, slab_last)],
                            acc_sh.at[pl.ds(sid * slab, slab_last)])

        plsc.subcore_barrier()

        def do_chunk(base, idx, b, k):
            pltpu.sync_copy(rows_hbm.at[pl.ds(base, k)], idx)
            pltpu.sync_copy(ef_hbm.at[pl.ds(base, k)], b)
            pltpu.sync_copy(b, acc_sh.at[idx], add=True)

        def chunk_loop(c, _):
            do_chunk(base0 + c * CHUNK, ir, buf, CHUNK)
            return 0

        lax.fori_loop(0, nfull, chunk_loop, 0)
        if rem:
            do_chunk(base0 + nfull * CHUNK, irt, buft, rem)
        plsc.subcore_barrier()

        @pl.when(sid < NS - 1)
        def _():
            pltpu.sync_copy(acc_sh.at[pl.ds(sid * slab, slab)],
                            out_hbm.at[cid, pl.ds(sid * slab, slab)])

        @pl.when(sid == NS - 1)
        def _():
            pltpu.sync_copy(acc_sh.at[pl.ds(sid * slab, slab_last)],
                            out_hbm.at[cid, pl.ds(sid * slab, slab_last)])

    return scatter_kernel(ef, rows, zeros)


def _node_body(h_ref, p0_ref, p1_ref, nm_ref, w1a_ref, w1b_ref, b1_ref,
               w2_ref, b2_ref, norm_inv_ref, out_ref):
    h = h_ref[...]
    agg = (p0_ref[...] + p1_ref[...]) * norm_inv_ref[0, 0]
    x = (jnp.dot(h, w1a_ref[...], preferred_element_type=jnp.float32)
         + jnp.dot(agg, w1b_ref[...], preferred_element_type=jnp.float32)
         + b1_ref[...])
    t = _silu(x)
    out = h + jnp.dot(t, w2_ref[...], preferred_element_type=jnp.float32) + b2_ref[...]
    out_ref[...] = out * nm_ref[...]


def _tc_node(h, p0, p1, node_mask, Wn1a, Wn1b, b_n1, W_n2, b_n2, norm):
    n, d = h.shape
    hdim = Wn1b.shape[0]
    bn = 2000
    grid = (n // bn,)
    b1 = b_n1.reshape(1, -1)
    b2 = b_n2.reshape(1, -1)
    norm_inv = jnp.full((1, 1), 1.0 / norm, jnp.float32)
    return pl.pallas_call(
        _node_body,
        grid=grid,
        in_specs=[
            pl.BlockSpec((bn, d), lambda i: (i, 0)),
            pl.BlockSpec((bn, hdim), lambda i: (i, 0)),
            pl.BlockSpec((bn, hdim), lambda i: (i, 0)),
            pl.BlockSpec((bn, 1), lambda i: (i, 0)),
            pl.BlockSpec(Wn1a.shape, lambda i: (0, 0)),
            pl.BlockSpec(Wn1b.shape, lambda i: (0, 0)),
            pl.BlockSpec((1, b_n1.shape[0]), lambda i: (0, 0)),
            pl.BlockSpec(W_n2.shape, lambda i: (0, 0)),
            pl.BlockSpec((1, b_n2.shape[0]), lambda i: (0, 0)),
            pl.BlockSpec((1, 1), lambda i: (0, 0)),
        ],
        out_specs=pl.BlockSpec((bn, d), lambda i: (i, 0)),
        out_shape=jax.ShapeDtypeStruct((n, d), jnp.float32),
    )(h, p0, p1, node_mask, Wn1a, Wn1b, b1, W_n2, b2, norm_inv)


def kernel(h, edge_index, edge_attr, node_mask, edge_mask,
           W_e1, b_e1, W_e2, b_e2, W_att, b_att,
           W_n1, b_n1, W_n2, b_n2):
    n, d = h.shape
    hdim = W_e2.shape[0]
    norm = 32.0
    WeA = W_e1[:d]
    WeB = W_e1[d:2 * d]
    WeC = W_e1[2 * d:]
    rows = edge_index[0]
    cols = edge_index[1]

    hA, hB = _tc_pre(h, WeA, WeB)
    pre1 = _sc_gather_add(hA, hB, rows, cols)
    mij, ef = _tc_edge(pre1, edge_attr, edge_mask, WeC, b_e1, W_e2, b_e2,
                       W_att, b_att)
    partials = _sc_scatter(ef, rows, n)
    h_out = _tc_node(h, partials[0], partials[1], node_mask,
                     W_n1[:d], W_n1[d:], b_n1, W_n2, b_n2, norm)
    return (h_out, mij)


# speedup vs baseline: 3.3998x; 1.3633x over previous
import functools

import jax
import jax.numpy as jnp
from jax import lax
from jax.experimental import pallas as pl
from jax.experimental.pallas import tpu as pltpu
from jax.experimental.pallas import tpu_sc as plsc

NC = 2
NS = 16
NW = NC * NS
LANES = 16
CHUNK = 128
TAIL = 16


def _silu(x):
    return x * jax.nn.sigmoid(x)


def _pre_body(h_ref, wa_ref, wb_ref, ha_ref, hb_ref):
    h = h_ref[...]
    ha_ref[...] = jnp.dot(h, wa_ref[...], preferred_element_type=jnp.float32)
    hb_ref[...] = jnp.dot(h, wb_ref[...], preferred_element_type=jnp.float32)


def _tc_pre(h, WeA, WeB):
    n, d = h.shape
    bn = 2000
    grid = (n // bn,)
    return pl.pallas_call(
        _pre_body,
        grid=grid,
        in_specs=[
            pl.BlockSpec((bn, d), lambda i: (i, 0)),
            pl.BlockSpec(WeA.shape, lambda i: (0, 0)),
            pl.BlockSpec(WeB.shape, lambda i: (0, 0)),
        ],
        out_specs=[
            pl.BlockSpec((bn, WeA.shape[1]), lambda i: (i, 0)),
            pl.BlockSpec((bn, WeB.shape[1]), lambda i: (i, 0)),
        ],
        out_shape=[
            jax.ShapeDtypeStruct((n, WeA.shape[1]), jnp.float32),
            jax.ShapeDtypeStruct((n, WeB.shape[1]), jnp.float32),
        ],
    )(h, WeA, WeB)


def _sc_gather_add(hA, hB, rows, cols):
    n, hdim = hA.shape
    e = rows.shape[0]
    ew = e // NW
    nfull = ew // CHUNK
    rem = ew - nfull * CHUNK
    mesh = plsc.VectorSubcoreMesh(
        core_axis_name="c", subcore_axis_name="s", num_cores=NC, num_subcores=NS)

    @functools.partial(
        pl.kernel,
        out_type=jax.ShapeDtypeStruct((e, hdim), jnp.float32),
        mesh=mesh,
        scratch_types=[
            pltpu.VMEM((CHUNK,), jnp.int32),
            pltpu.VMEM((CHUNK,), jnp.int32),
            pltpu.VMEM((CHUNK, hdim), jnp.float32),
            pltpu.VMEM((CHUNK, hdim), jnp.float32),
            pltpu.VMEM((TAIL,), jnp.int32),
            pltpu.VMEM((TAIL,), jnp.int32),
            pltpu.VMEM((TAIL, hdim), jnp.float32),
            pltpu.VMEM((TAIL, hdim), jnp.float32),
            pltpu.SemaphoreType.DMA,
            pltpu.SemaphoreType.DMA,
        ],
    )
    def gather_kernel(ha_hbm, hb_hbm, rows_hbm, cols_hbm, out_hbm,
                      ir, ic, ba, bb, irt, ict, bat, bbt, sa, sb):
        wid = lax.axis_index("s") * NC + lax.axis_index("c")
        base0 = wid * ew

        def do_chunk(base, idxr, idxc, bufa, bufb, k):
            pltpu.sync_copy(rows_hbm.at[pl.ds(base, k)], idxr)
            pltpu.sync_copy(cols_hbm.at[pl.ds(base, k)], idxc)
            cpa = pltpu.async_copy(ha_hbm.at[idxr], bufa, sa)
            cpb = pltpu.async_copy(hb_hbm.at[idxc], bufb, sb)
            cpa.wait()
            cpb.wait()

            def addrow(i, _):
                for j in range(hdim // LANES):
                    sl = pl.ds(j * LANES, LANES)
                    bufa[i, sl] = bufa[i, sl] + bufb[i, sl]
                return 0

            lax.fori_loop(0, k, addrow, 0)
            pltpu.sync_copy(bufa, out_hbm.at[pl.ds(base, k)])

        def chunk_loop(c, _):
            do_chunk(base0 + c * CHUNK, ir, ic, ba, bb, CHUNK)
            return 0

        lax.fori_loop(0, nfull, chunk_loop, 0)
        if rem:
            do_chunk(base0 + nfull * CHUNK, irt, ict, bat, bbt, rem)

    return gather_kernel(hA, hB, rows, cols)


def _edge_body(pre_ref, ea_ref, em_ref, wc_ref, b1_ref, w2_ref, b2_ref,
               wa_ref, ba_ref, mij_ref, ef_ref):
    x = pre_ref[...] + jnp.dot(ea_ref[...], wc_ref[...],
                               preferred_element_type=jnp.float32) + b1_ref[...]
    m = _silu(x)
    y = jnp.dot(m, w2_ref[...], preferred_element_type=jnp.float32) + b2_ref[...]
    mij = _silu(y)
    att = jax.nn.sigmoid(
        jnp.sum(mij * wa_ref[...], axis=1, keepdims=True) + ba_ref[0, 0])
    mij_ref[...] = mij
    ef_ref[...] = mij * (att * em_ref[...])


def _tc_edge(pre1, edge_attr, edge_mask, WeC, b_e1, W_e2, b_e2, W_att, b_att):
    e, hdim = pre1.shape
    de = edge_attr.shape[1]
    be = 6400
    grid = (e // be,)
    wa_row = W_att.reshape(1, hdim)
    ba = b_att.reshape(1, 1)
    b1 = b_e1.reshape(1, hdim)
    b2 = b_e2.reshape(1, hdim)
    return pl.pallas_call(
        _edge_body,
        grid=grid,
        in_specs=[
            pl.BlockSpec((be, hdim), lambda i: (i, 0)),
            pl.BlockSpec((be, de), lambda i: (i, 0)),
            pl.BlockSpec((be, 1), lambda i: (i, 0)),
            pl.BlockSpec((de, hdim), lambda i: (0, 0)),
            pl.BlockSpec((1, hdim), lambda i: (0, 0)),
            pl.BlockSpec((hdim, hdim), lambda i: (0, 0)),
            pl.BlockSpec((1, hdim), lambda i: (0, 0)),
            pl.BlockSpec((1, hdim), lambda i: (0, 0)),
            pl.BlockSpec((1, 1), lambda i: (0, 0)),
        ],
        out_specs=[
            pl.BlockSpec((be, hdim), lambda i: (i, 0)),
            pl.BlockSpec((be, hdim), lambda i: (i, 0)),
        ],
        out_shape=[
            jax.ShapeDtypeStruct((e, hdim), jnp.float32),
            jax.ShapeDtypeStruct((e, hdim), jnp.float32),
        ],
    )(pre1, edge_attr, edge_mask, WeC, b1, W_e2, b2, wa_row, ba)


def _sc_scatter(ef, rows, n):
    e, hdim = ef.shape
    ew = e // NW
    nfull = ew // CHUNK
    rem = ew - nfull * CHUNK
    slab = ((n + NS - 1) // NS + 7) // 8 * 8
    slab_last = n - slab * (NS - 1)
    assert slab_last > 0
    mesh = plsc.VectorSubcoreMesh(
        core_axis_name="c", subcore_axis_name="s", num_cores=NC, num_subcores=NS)
    zeros = jnp.zeros((slab, hdim), jnp.float32)

    @functools.partial(
        pl.kernel,
        out_type=jax.ShapeDtypeStruct((NC, n, hdim), jnp.float32),
        mesh=mesh,
        scratch_types=[
            pltpu.VMEM((CHUNK,), jnp.int32),
            pltpu.VMEM((CHUNK, hdim), jnp.float32),
            pltpu.VMEM((TAIL,), jnp.int32),
            pltpu.VMEM((TAIL, hdim), jnp.float32),
            pltpu.VMEM_SHARED((n, hdim), jnp.float32),
        ],
    )
    def scatter_kernel(ef_hbm, rows_hbm, z_hbm, out_hbm, ir, buf, irt, buft, acc_sh):
        cid = lax.axis_index("c")
        sid = lax.axis_index("s")
        wid = sid * NC + cid
        base0 = wid * ew

        @pl.when(sid < NS - 1)
        def _():
            pltpu.sync_copy(z_hbm, acc_sh.at[pl.ds(sid * slab, slab)])

        @pl.when(sid == NS - 1)
        def _():
            pltpu.sync_copy(z_hbm.at[pl.ds(0, slab_last)],
                            acc_sh.at[pl.ds(sid * slab, slab_last)])

        plsc.subcore_barrier()

        def do_chunk(base, idx, b, k):
            pltpu.sync_copy(rows_hbm.at[pl.ds(base, k)], idx)
            pltpu.sync_copy(ef_hbm.at[pl.ds(base, k)], b)
            pltpu.sync_copy(b, acc_sh.at[idx], add=True)

        def chunk_loop(c, _):
            do_chunk(base0 + c * CHUNK, ir, buf, CHUNK)
            return 0

        lax.fori_loop(0, nfull, chunk_loop, 0)
        if rem:
            do_chunk(base0 + nfull * CHUNK, irt, buft, rem)
        plsc.subcore_barrier()

        @pl.when(sid < NS - 1)
        def _():
            pltpu.sync_copy(acc_sh.at[pl.ds(sid * slab, slab)],
                            out_hbm.at[cid, pl.ds(sid * slab, slab)])

        @pl.when(sid == NS - 1)
        def _():
            pltpu.sync_copy(acc_sh.at[pl.ds(sid * slab, slab_last)],
                            out_hbm.at[cid, pl.ds(sid * slab, slab_last)])

    return scatter_kernel(ef, rows, zeros)


def _node_body(h_ref, p0_ref, p1_ref, nm_ref, w1a_ref, w1b_ref, b1_ref,
               w2_ref, b2_ref, norm_inv_ref, out_ref):
    h = h_ref[...]
    agg = (p0_ref[...] + p1_ref[...]) * norm_inv_ref[0, 0]
    x = (jnp.dot(h, w1a_ref[...], preferred_element_type=jnp.float32)
         + jnp.dot(agg, w1b_ref[...], preferred_element_type=jnp.float32)
         + b1_ref[...])
    t = _silu(x)
    out = h + jnp.dot(t, w2_ref[...], preferred_element_type=jnp.float32) + b2_ref[...]
    out_ref[...] = out * nm_ref[...]


def _tc_node(h, p0, p1, node_mask, Wn1a, Wn1b, b_n1, W_n2, b_n2, norm):
    n, d = h.shape
    hdim = Wn1b.shape[0]
    bn = 2000
    grid = (n // bn,)
    b1 = b_n1.reshape(1, -1)
    b2 = b_n2.reshape(1, -1)
    norm_inv = jnp.full((1, 1), 1.0 / norm, jnp.float32)
    return pl.pallas_call(
        _node_body,
        grid=grid,
        in_specs=[
            pl.BlockSpec((bn, d), lambda i: (i, 0)),
            pl.BlockSpec((bn, hdim), lambda i: (i, 0)),
            pl.BlockSpec((bn, hdim), lambda i: (i, 0)),
            pl.BlockSpec((bn, 1), lambda i: (i, 0)),
            pl.BlockSpec(Wn1a.shape, lambda i: (0, 0)),
            pl.BlockSpec(Wn1b.shape, lambda i: (0, 0)),
            pl.BlockSpec((1, b_n1.shape[0]), lambda i: (0, 0)),
            pl.BlockSpec(W_n2.shape, lambda i: (0, 0)),
            pl.BlockSpec((1, b_n2.shape[0]), lambda i: (0, 0)),
            pl.BlockSpec((1, 1), lambda i: (0, 0)),
        ],
        out_specs=pl.BlockSpec((bn, d), lambda i: (i, 0)),
        out_shape=jax.ShapeDtypeStruct((n, d), jnp.float32),
    )(h, p0, p1, node_mask, Wn1a, Wn1b, b1, W_n2, b2, norm_inv)


def kernel(h, edge_index, edge_attr, node_mask, edge_mask,
           W_e1, b_e1, W_e2, b_e2, W_att, b_att,
           W_n1, b_n1, W_n2, b_n2):
    n, d = h.shape
    hdim = W_e2.shape[0]
    norm = 32.0
    WeA = W_e1[:d]
    WeB = W_e1[d:2 * d]
    WeC = W_e1[2 * d:]
    rows = edge_index[0]
    cols = edge_index[1]

    hA, hB = _tc_pre(h, WeA, WeB)
    pre1 = _sc_gather_add(hA, hB, rows, cols)
    mij, ef = _tc_edge(pre1, edge_attr, edge_mask, WeC, b_e1, W_e2, b_e2,
                       W_att, b_att)
    partials = _sc_scatter(ef, rows, n)
    h_out = _tc_node(h, partials[0], partials[1], node_mask,
                     W_n1[:d], W_n1[d:], b_n1, W_n2, b_n2, norm)
    return (h_out, mij)
